# baseline (speedup 1.0000x reference)
import jax
import jax.numpy as jnp
from jax import lax
from jax.experimental import pallas as pl
from jax.experimental.pallas import tpu as pltpu

N = 10000
E = 320000
D = 128
NCORE = 2
EB = 16000
NBE = E // NCORE // EB
BLK = 1000
BANKS = 8
RB = 500


def _mlp1_body(x_ref, wa_ref, wb_ref, y_ref):
    h = jnp.maximum(
        jnp.dot(x_ref[...], wa_ref[...], preferred_element_type=jnp.float32), 0.0)
    y_ref[...] = jnp.dot(h, wb_ref[...], preferred_element_type=jnp.float32)


def _mlp1(x, W1a, W1b):
    return pl.pallas_call(
        _mlp1_body,
        grid=(N // BLK,),
        in_specs=[
            pl.BlockSpec((BLK, D), lambda i: (i, 0)),
            pl.BlockSpec((D, D), lambda i: (0, 0)),
            pl.BlockSpec((D, D), lambda i: (0, 0)),
        ],
        out_specs=pl.BlockSpec((BLK, D), lambda i: (i, 0)),
        out_shape=jax.ShapeDtypeStruct((N, D), jnp.float32),
    )(x, W1a, W1b)


def _scatter_body(idx_ref, y_ref, sum_ref, cnt_ref, *scratch):
    sbanks = scratch[:BANKS - 1]
    cbank = scratch[BANKS - 1]
    j = pl.program_id(1)

    @pl.when(j == 0)
    def _():
        sum_ref[...] = jnp.zeros((1, N, D), jnp.float32)
        for s in sbanks:
            s[...] = jnp.zeros((N, D), jnp.float32)

        def _cz(n, carry):
            cnt_ref[0, 0, n] = 0
            cbank[n] = 0
            return carry

        lax.fori_loop(0, N, _cz, 0)

    def _edges(e, carry):
        base = e * BANKS
        for k in range(BANKS):
            r = idx_ref[0, base + k]
            c = idx_ref[1, base + k]
            msg = y_ref[pl.ds(r, 1), :]
            if k == 0:
                sum_ref[0, pl.ds(c, 1), :] = sum_ref[0, pl.ds(c, 1), :] + msg
            else:
                sbanks[k - 1][pl.ds(c, 1), :] = sbanks[k - 1][pl.ds(c, 1), :] + msg
            if k % 2 == 0:
                cnt_ref[0, 0, c] = cnt_ref[0, 0, c] + 1
            else:
                cbank[c] = cbank[c] + 1
        return carry

    lax.fori_loop(0, EB // BANKS, _edges, 0)

    @pl.when(j == NBE - 1)
    def _():
        def _red(i, carry):
            o = i * RB
            acc = sum_ref[0, pl.ds(o, RB), :]
            for s in sbanks:
                acc = acc + s[pl.ds(o, RB), :]
            sum_ref[0, pl.ds(o, RB), :] = acc
            return carry

        lax.fori_loop(0, N // RB, _red, 0)

        def _cred(n, carry):
            cnt_ref[0, 0, n] = cnt_ref[0, 0, n] + cbank[n]
            return carry

        lax.fori_loop(0, N, _cred, 0)


def _scatter_tc(edge_index, y):
    return pl.pallas_call(
        _scatter_body,
        grid=(NCORE, NBE),
        in_specs=[
            pl.BlockSpec((2, EB), lambda i, j: (0, i * NBE + j),
                         memory_space=pltpu.SMEM),
            pl.BlockSpec((N, D), lambda i, j: (0, 0)),
        ],
        out_specs=[
            pl.BlockSpec((1, N, D), lambda i, j: (i, 0, 0)),
            pl.BlockSpec((1, 1, N), lambda i, j: (i, 0, 0),
                         memory_space=pltpu.SMEM),
        ],
        out_shape=[
            jax.ShapeDtypeStruct((NCORE, N, D), jnp.float32),
            jax.ShapeDtypeStruct((NCORE, 1, N), jnp.int32),
        ],
        scratch_shapes=(
            [pltpu.VMEM((N, D), jnp.float32)] * (BANKS - 1)
            + [pltpu.SMEM((N,), jnp.int32)]
        ),
        compiler_params=pltpu.CompilerParams(
            dimension_semantics=("parallel", "arbitrary")),
    )(edge_index, y)


def _post_body(x_ref, a_ref, c1_ref, c2_ref, w2a_ref, w2b_ref,
               w_ref, g1_ref, b1_ref, g2_ref, b2_ref, o_ref):
    s = a_ref[0] + a_ref[1]
    cnt = c1_ref[...] + c2_ref[...]
    agg = s / jnp.maximum(cnt, 1.0)
    mu = jnp.mean(agg, axis=-1, keepdims=True)
    var = jnp.mean((agg - mu) ** 2, axis=-1, keepdims=True)
    agg = (agg - mu) / jnp.sqrt(var + 1e-5) * g1_ref[...] + b1_ref[...]
    x = x_ref[...]
    fx = x + (x - agg) * w_ref[...]
    mu2 = jnp.mean(fx, axis=-1, keepdims=True)
    var2 = jnp.mean((fx - mu2) ** 2, axis=-1, keepdims=True)
    fx = (fx - mu2) / jnp.sqrt(var2 + 1e-5) * g2_ref[...] + b2_ref[...]
    h = jnp.maximum(
        jnp.dot(fx, w2a_ref[:D], preferred_element_type=jnp.float32)
        + jnp.dot(agg, w2a_ref[D:], preferred_element_type=jnp.float32), 0.0)
    o_ref[...] = jnp.dot(h, w2b_ref[...], preferred_element_type=jnp.float32)


def _post(x, a, c1, c2, W2a, W2b, w, g1, b1, g2, b2):
    vec = pl.BlockSpec((1, D), lambda i: (0, 0))
    return pl.pallas_call(
        _post_body,
        grid=(N // BLK,),
        in_specs=[
            pl.BlockSpec((BLK, D), lambda i: (i, 0)),
            pl.BlockSpec((NCORE, BLK, D), lambda i: (0, i, 0)),
            pl.BlockSpec((BLK, 1), lambda i: (i, 0)),
            pl.BlockSpec((BLK, 1), lambda i: (i, 0)),
            pl.BlockSpec((2 * D, D), lambda i: (0, 0)),
            pl.BlockSpec((D, D), lambda i: (0, 0)),
            vec, vec, vec, vec, vec,
        ],
        out_specs=pl.BlockSpec((BLK, D), lambda i: (i, 0)),
        out_shape=jax.ShapeDtypeStruct((N, D), jnp.float32),
    )(x, a, c1, c2, W2a, W2b, w.reshape(1, D), g1.reshape(1, D),
      b1.reshape(1, D), g2.reshape(1, D), b2.reshape(1, D))


def kernel(x, edge_index, W1a, W1b, W2a, W2b, w, ln1_g, ln1_b, ln2_g, ln2_b):
    y = _mlp1(x, W1a, W1b)
    s, cnt = _scatter_tc(edge_index, y)
    c1 = cnt[0].astype(jnp.float32).reshape(N, 1)
    c2 = cnt[1].astype(jnp.float32).reshape(N, 1)
    return _post(x, s, c1, c2, W2a, W2b, w, ln1_g, ln1_b, ln2_g, ln2_b)


# speedup vs baseline: 1.2106x; 1.2106x over previous
import jax
import jax.numpy as jnp
from jax import lax
from jax.experimental import pallas as pl
from jax.experimental.pallas import tpu as pltpu

N = 10000
E = 320000
D = 128
NCORE = 4
EB = 16000
NBE = E // NCORE // EB
BLK = 1000
BANKS = 8
CR = 80
RB = 500


def _mlp1_body(x_ref, wa_ref, wb_ref, y_ref):
    h = jnp.maximum(
        jnp.dot(x_ref[...], wa_ref[...], preferred_element_type=jnp.float32), 0.0)
    y_ref[...] = jnp.dot(h, wb_ref[...], preferred_element_type=jnp.float32)


def _mlp1(x, W1a, W1b):
    return pl.pallas_call(
        _mlp1_body,
        grid=(N // BLK,),
        in_specs=[
            pl.BlockSpec((BLK, D), lambda i: (i, 0)),
            pl.BlockSpec((D, D), lambda i: (0, 0)),
            pl.BlockSpec((D, D), lambda i: (0, 0)),
        ],
        out_specs=pl.BlockSpec((BLK, D), lambda i: (i, 0)),
        out_shape=jax.ShapeDtypeStruct((N, D), jnp.float32),
    )(x, W1a, W1b)


def _scatter_body(idx_ref, y_ref, sum_ref, cnt_ref, *scratch):
    sbanks = scratch[:BANKS - 1]
    cbanks = scratch[BANKS - 1:]
    j = pl.program_id(1)

    @pl.when(j == 0)
    def _():
        sum_ref[...] = jnp.zeros((1, N, D), jnp.float32)
        cnt_ref[...] = jnp.zeros((1, CR, D), jnp.float32)
        for s in sbanks:
            s[...] = jnp.zeros((N, D), jnp.float32)
        for c in cbanks:
            c[...] = jnp.zeros((CR, D), jnp.float32)

    lane = lax.broadcasted_iota(jnp.int32, (1, D), 1)

    def _edges(e, carry):
        base = e * BANKS
        for k in range(BANKS):
            r = idx_ref[0, base + k]
            c = idx_ref[1, base + k]
            msg = y_ref[pl.ds(r, 1), :]
            cr = lax.shift_right_logical(c, 7)
            hot = (lane == jnp.bitwise_and(c, 127)).astype(jnp.float32)
            if k == 0:
                sum_ref[0, pl.ds(c, 1), :] = sum_ref[0, pl.ds(c, 1), :] + msg
                cnt_ref[0, pl.ds(cr, 1), :] = cnt_ref[0, pl.ds(cr, 1), :] + hot
            else:
                sbanks[k - 1][pl.ds(c, 1), :] = sbanks[k - 1][pl.ds(c, 1), :] + msg
                cbanks[k - 1][pl.ds(cr, 1), :] = cbanks[k - 1][pl.ds(cr, 1), :] + hot
        return carry

    lax.fori_loop(0, EB // BANKS, _edges, 0)

    @pl.when(j == NBE - 1)
    def _():
        def _red(i, carry):
            o = i * RB
            acc = sum_ref[0, pl.ds(o, RB), :]
            for s in sbanks:
                acc = acc + s[pl.ds(o, RB), :]
            sum_ref[0, pl.ds(o, RB), :] = acc
            return carry

        lax.fori_loop(0, N // RB, _red, 0)
        cacc = cnt_ref[0]
        for c in cbanks:
            cacc = cacc + c[...]
        cnt_ref[0] = cacc


def _scatter_tc(edge_index, y):
    return pl.pallas_call(
        _scatter_body,
        grid=(NCORE, NBE),
        in_specs=[
            pl.BlockSpec((2, EB), lambda i, j: (0, i * NBE + j),
                         memory_space=pltpu.SMEM),
            pl.BlockSpec((N, D), lambda i, j: (0, 0)),
        ],
        out_specs=[
            pl.BlockSpec((1, N, D), lambda i, j: (i, 0, 0)),
            pl.BlockSpec((1, CR, D), lambda i, j: (i, 0, 0)),
        ],
        out_shape=[
            jax.ShapeDtypeStruct((NCORE, N, D), jnp.float32),
            jax.ShapeDtypeStruct((NCORE, CR, D), jnp.float32),
        ],
        scratch_shapes=(
            [pltpu.VMEM((N, D), jnp.float32)] * (BANKS - 1)
            + [pltpu.VMEM((CR, D), jnp.float32)] * (BANKS - 1)
        ),
        compiler_params=pltpu.CompilerParams(
            dimension_semantics=("parallel", "arbitrary")),
    )(edge_index, y)


def _post_body(x_ref, a_ref, c_ref, w2a_ref, w2b_ref,
               w_ref, g1_ref, b1_ref, g2_ref, b2_ref, o_ref):
    s = jnp.sum(a_ref[...], axis=0)
    cnt = jnp.sum(c_ref[...], axis=0)
    agg = s / jnp.maximum(cnt, 1.0)
    mu = jnp.mean(agg, axis=-1, keepdims=True)
    var = jnp.mean((agg - mu) ** 2, axis=-1, keepdims=True)
    agg = (agg - mu) / jnp.sqrt(var + 1e-5) * g1_ref[...] + b1_ref[...]
    x = x_ref[...]
    fx = x + (x - agg) * w_ref[...]
    mu2 = jnp.mean(fx, axis=-1, keepdims=True)
    var2 = jnp.mean((fx - mu2) ** 2, axis=-1, keepdims=True)
    fx = (fx - mu2) / jnp.sqrt(var2 + 1e-5) * g2_ref[...] + b2_ref[...]
    h = jnp.maximum(
        jnp.dot(fx, w2a_ref[:D], preferred_element_type=jnp.float32)
        + jnp.dot(agg, w2a_ref[D:], preferred_element_type=jnp.float32), 0.0)
    o_ref[...] = jnp.dot(h, w2b_ref[...], preferred_element_type=jnp.float32)


def _post(x, a, c, W2a, W2b, w, g1, b1, g2, b2):
    vec = pl.BlockSpec((1, D), lambda i: (0, 0))
    return pl.pallas_call(
        _post_body,
        grid=(N // BLK,),
        in_specs=[
            pl.BlockSpec((BLK, D), lambda i: (i, 0)),
            pl.BlockSpec((NCORE, BLK, D), lambda i: (0, i, 0)),
            pl.BlockSpec((NCORE, BLK, 1), lambda i: (0, i, 0)),
            pl.BlockSpec((2 * D, D), lambda i: (0, 0)),
            pl.BlockSpec((D, D), lambda i: (0, 0)),
            vec, vec, vec, vec, vec,
        ],
        out_specs=pl.BlockSpec((BLK, D), lambda i: (i, 0)),
        out_shape=jax.ShapeDtypeStruct((N, D), jnp.float32),
    )(x, a, c, W2a, W2b, w.reshape(1, D), g1.reshape(1, D),
      b1.reshape(1, D), g2.reshape(1, D), b2.reshape(1, D))


def kernel(x, edge_index, W1a, W1b, W2a, W2b, w, ln1_g, ln1_b, ln2_g, ln2_b):
    y = _mlp1(x, W1a, W1b)
    s, cnt = _scatter_tc(edge_index, y)
    c = cnt.reshape(NCORE, -1)[:, :N].reshape(NCORE, N, 1)
    return _post(x, s, c, W2a, W2b, w, ln1_g, ln1_b, ln2_g, ln2_b)


# speedup vs baseline: 1.2263x; 1.0130x over previous
import jax
import jax.numpy as jnp
from jax import lax
from jax.experimental import pallas as pl
from jax.experimental.pallas import tpu as pltpu

N = 10000
E = 320000
D = 128
NCORE = 2
EB = 16000
NBE = E // NCORE // EB
BLK = 1000
BANKS = 8
CR = 80
RB = 500


def _mlp1_body(x_ref, wa_ref, wb_ref, y_ref):
    h = jnp.maximum(
        jnp.dot(x_ref[...], wa_ref[...], preferred_element_type=jnp.float32), 0.0)
    y_ref[...] = jnp.dot(h, wb_ref[...], preferred_element_type=jnp.float32)


def _mlp1(x, W1a, W1b):
    return pl.pallas_call(
        _mlp1_body,
        grid=(N // BLK,),
        in_specs=[
            pl.BlockSpec((BLK, D), lambda i: (i, 0)),
            pl.BlockSpec((D, D), lambda i: (0, 0)),
            pl.BlockSpec((D, D), lambda i: (0, 0)),
        ],
        out_specs=pl.BlockSpec((BLK, D), lambda i: (i, 0)),
        out_shape=jax.ShapeDtypeStruct((N, D), jnp.float32),
    )(x, W1a, W1b)


def _scatter_body(idx_ref, y_ref, sum_ref, cnt_ref, *scratch):
    sbanks = scratch[:BANKS - 1]
    cbanks = scratch[BANKS - 1:]
    j = pl.program_id(1)

    @pl.when(j == 0)
    def _():
        sum_ref[...] = jnp.zeros((1, N, D), jnp.float32)
        cnt_ref[...] = jnp.zeros((1, CR, D), jnp.float32)
        for s in sbanks:
            s[...] = jnp.zeros((N, D), jnp.float32)
        for c in cbanks:
            c[...] = jnp.zeros((CR, D), jnp.float32)

    lane = lax.broadcasted_iota(jnp.int32, (1, D), 1)

    def _edges(e, carry):
        base = e * BANKS
        for k in range(BANKS):
            r = idx_ref[0, base + k]
            c = idx_ref[1, base + k]
            msg = y_ref[pl.ds(r, 1), :]
            cr = lax.shift_right_logical(c, 7)
            hot = (lane == jnp.bitwise_and(c, 127)).astype(jnp.float32)
            if k == 0:
                sum_ref[0, pl.ds(c, 1), :] = sum_ref[0, pl.ds(c, 1), :] + msg
                cnt_ref[0, pl.ds(cr, 1), :] = cnt_ref[0, pl.ds(cr, 1), :] + hot
            else:
                sbanks[k - 1][pl.ds(c, 1), :] = sbanks[k - 1][pl.ds(c, 1), :] + msg
                cbanks[k - 1][pl.ds(cr, 1), :] = cbanks[k - 1][pl.ds(cr, 1), :] + hot
        return carry

    lax.fori_loop(0, EB // BANKS, _edges, 0)

    @pl.when(j == NBE - 1)
    def _():
        def _red(i, carry):
            o = i * RB
            acc = sum_ref[0, pl.ds(o, RB), :]
            for s in sbanks:
                acc = acc + s[pl.ds(o, RB), :]
            sum_ref[0, pl.ds(o, RB), :] = acc
            return carry

        lax.fori_loop(0, N // RB, _red, 0)
        cacc = cnt_ref[0]
        for c in cbanks:
            cacc = cacc + c[...]
        cnt_ref[0] = cacc


def _scatter_tc(edge_index, y):
    return pl.pallas_call(
        _scatter_body,
        grid=(NCORE, NBE),
        in_specs=[
            pl.BlockSpec((2, EB), lambda i, j: (0, i * NBE + j),
                         memory_space=pltpu.SMEM),
            pl.BlockSpec((N, D), lambda i, j: (0, 0)),
        ],
        out_specs=[
            pl.BlockSpec((1, N, D), lambda i, j: (i, 0, 0)),
            pl.BlockSpec((1, CR, D), lambda i, j: (i, 0, 0)),
        ],
        out_shape=[
            jax.ShapeDtypeStruct((NCORE, N, D), jnp.float32),
            jax.ShapeDtypeStruct((NCORE, CR, D), jnp.float32),
        ],
        scratch_shapes=(
            [pltpu.VMEM((N, D), jnp.float32)] * (BANKS - 1)
            + [pltpu.VMEM((CR, D), jnp.float32)] * (BANKS - 1)
        ),
        compiler_params=pltpu.CompilerParams(
            dimension_semantics=("parallel", "arbitrary")),
    )(edge_index, y)


def _post_body(x_ref, a_ref, c_ref, w2a_ref, w2b_ref,
               w_ref, g1_ref, b1_ref, g2_ref, b2_ref, o_ref):
    s = jnp.sum(a_ref[...], axis=0)
    cnt = jnp.sum(c_ref[...], axis=0)
    agg = s / jnp.maximum(cnt, 1.0)
    mu = jnp.mean(agg, axis=-1, keepdims=True)
    var = jnp.mean((agg - mu) ** 2, axis=-1, keepdims=True)
    agg = (agg - mu) / jnp.sqrt(var + 1e-5) * g1_ref[...] + b1_ref[...]
    x = x_ref[...]
    fx = x + (x - agg) * w_ref[...]
    mu2 = jnp.mean(fx, axis=-1, keepdims=True)
    var2 = jnp.mean((fx - mu2) ** 2, axis=-1, keepdims=True)
    fx = (fx - mu2) / jnp.sqrt(var2 + 1e-5) * g2_ref[...] + b2_ref[...]
    h = jnp.maximum(
        jnp.dot(fx, w2a_ref[:D], preferred_element_type=jnp.float32)
        + jnp.dot(agg, w2a_ref[D:], preferred_element_type=jnp.float32), 0.0)
    o_ref[...] = jnp.dot(h, w2b_ref[...], preferred_element_type=jnp.float32)


def _post(x, a, c, W2a, W2b, w, g1, b1, g2, b2):
    vec = pl.BlockSpec((1, D), lambda i: (0, 0))
    return pl.pallas_call(
        _post_body,
        grid=(N // BLK,),
        in_specs=[
            pl.BlockSpec((BLK, D), lambda i: (i, 0)),
            pl.BlockSpec((NCORE, BLK, D), lambda i: (0, i, 0)),
            pl.BlockSpec((NCORE, BLK, 1), lambda i: (0, i, 0)),
            pl.BlockSpec((2 * D, D), lambda i: (0, 0)),
            pl.BlockSpec((D, D), lambda i: (0, 0)),
            vec, vec, vec, vec, vec,
        ],
        out_specs=pl.BlockSpec((BLK, D), lambda i: (i, 0)),
        out_shape=jax.ShapeDtypeStruct((N, D), jnp.float32),
    )(x, a, c, W2a, W2b, w.reshape(1, D), g1.reshape(1, D),
      b1.reshape(1, D), g2.reshape(1, D), b2.reshape(1, D))


def kernel(x, edge_index, W1a, W1b, W2a, W2b, w, ln1_g, ln1_b, ln2_g, ln2_b):
    y = _mlp1(x, W1a, W1b)
    s, cnt = _scatter_tc(edge_index, y)
    c = cnt.reshape(NCORE, -1)[:, :N].reshape(NCORE, N, 1)
    return _post(x, s, c, W2a, W2b, w, ln1_g, ln1_b, ln2_g, ln2_b)
